# baseline (speedup 1.0000x reference)
import functools

import jax
import jax.numpy as jnp
from jax import lax
from jax.experimental import pallas as pl
from jax.experimental.pallas import tpu as pltpu
from jax.experimental.pallas import tpu_sc as plsc

NC = 2
NS = 16
NW = NC * NS
L = 16

W = 4096


def _lane_i(v, j):
    return jnp.sum(jnp.where(lax.iota(jnp.int32, L) == j, v, 0))


def _lane_f(v, j):
    return jnp.sum(jnp.where(lax.iota(jnp.int32, L) == j, v, jnp.float32(0.0)))


def _recip(x):
    xi = lax.bitcast_convert_type(x, jnp.int32)
    y = lax.bitcast_convert_type(jnp.int32(0x7EF477D5) - xi, jnp.float32)
    for _ in range(3):
        y = y * (jnp.float32(2.0) - x * y)
    return y


def _round_scalar(x):
    return (x + jnp.float32(0.5) * jnp.sign(x)).astype(jnp.int32).astype(
        jnp.float32)


def _sc_body(nlig, flt, pos_hbm, idx_hbm, par_hbm, u_hbm, out_hbm,
             idxv, idx96, vals, subi, subp, ubuf, zbuf,
             sem_z, sem_g, sem_s):
    cid = lax.axis_index("c")
    sid = lax.axis_index("s")
    w = sid * NC + cid

    n96 = 3 * nlig
    nch = n96 // L
    full_wins = flt // W
    tail = flt - full_wins * W
    tail_owner = full_wins % NW
    iota = lax.iota(jnp.int32, L)

    pltpu.sync_copy(idx_hbm, idxv)
    pltpu.sync_copy(par_hbm, ubuf)

    def _zero(g, c):
        zbuf[pl.ds(g * L, L)] = jnp.zeros((L,), jnp.float32)
        return c

    lax.fori_loop(0, W // L, _zero, 0)

    nwin = (full_wins - w + NW - 1) // NW

    def _fire(kk, c):
        win = w + kk * NW
        pltpu.async_copy(zbuf, out_hbm.at[pl.ds(win * W, W)], sem_z)
        return c

    lax.fori_loop(0, nwin, _fire, 0)

    if tail > 0:
        @pl.when(w == tail_owner)
        def _():
            pltpu.async_copy(zbuf.at[pl.ds(0, tail)],
                             out_hbm.at[pl.ds(full_wins * W, tail)], sem_z)

    for c in range(nch):
        e = iota + c * L
        row = lax.div(e, jnp.int32(3))
        comp = lax.rem(e, jnp.int32(3))
        g = plsc.load_gather(idxv, [row])
        idx96[pl.ds(c * L, L)] = g * 3 + comp

    pltpu.async_copy(pos_hbm.at[idx96], vals, sem_g).wait()

    if True:
        pv = ubuf[...]
        s = []
        for comp in range(3):
            acc = jnp.float32(0.0)
            for c in range(nch):
                e = iota + c * L
                m = lax.rem(e, jnp.int32(3)) == comp
                v = vals[pl.ds(c * L, L)]
                acc = acc + jnp.sum(jnp.where(m, v, jnp.float32(0.0)))
            s.append(acc)
        com = [sc * jnp.float32(1.0 / nlig) for sc in s]

        b = [[_lane_f(pv, 3 * i + j) for j in range(3)] for i in range(3)]
        r = [_lane_f(pv, 9 + j) for j in range(3)]
        kk = _lane_f(pv, 12)

    if True:
        d = [com[j] - r[j] for j in range(3)]
        for i in (2, 1, 0):
            sc_i = _round_scalar(d[i] * _recip(b[i][i]))
            d = [d[j] - sc_i * b[i][j] for j in range(3)]
        U = kk * (d[0] * d[0] + d[1] * d[1] + d[2] * d[2])
        F = [jnp.float32(-2.0 / nlig) * kk * d[j] for j in range(3)]
    cnt = jnp.int32(0)
    if True:
        for c in range(nch):
            ft = idx96[pl.ds(c * L, L)]
            m = lax.rem(lax.div(ft, jnp.int32(W)), jnp.int32(NW)) == w
            pos = jnp.cumsum(m.astype(jnp.int32)) + cnt - 1
            pos = jnp.maximum(pos, 0)
            e = iota + c * L
            comp = lax.rem(e, jnp.int32(3))
            val = jnp.where(comp == 0, F[0],
                            jnp.where(comp == 1, F[1], F[2]))
            plsc.store_scatter(subi, [pos], ft, mask=m)
            plsc.store_scatter(subp, [pos], val, mask=m)
            cnt = cnt + jnp.sum(m.astype(jnp.int32))

    def _drain(kk_, c):
        win = w + kk_ * NW
        pltpu.make_async_copy(zbuf, out_hbm.at[pl.ds(win * W, W)],
                              sem_z).wait()
        return c

    lax.fori_loop(0, nwin, _drain, 0)

    if tail > 0:
        @pl.when(w == tail_owner)
        def _():
            pltpu.make_async_copy(zbuf.at[pl.ds(0, tail)],
                                  out_hbm.at[pl.ds(full_wins * W, tail)],
                                  sem_z).wait()

    if True:
        @pl.when(cnt > 0)
        def _():
            fi = _lane_i(subi[pl.ds(0, L)], 0)
            fp = _lane_f(subp[pl.ds(0, L)], 0)
            for c in range(nch):
                e = iota + c * L
                wm = e < cnt
                si = subi[pl.ds(c * L, L)]
                sp = subp[pl.ds(c * L, L)]
                subi[pl.ds(c * L, L)] = jnp.where(wm, si, fi)
                subp[pl.ds(c * L, L)] = jnp.where(wm, sp, fp)
            pltpu.async_copy(subp, out_hbm.at[subi], sem_s).wait()

    @pl.when(w == 0)
    def _():
        ubuf[...] = jnp.where(iota == 0, U, jnp.float32(0.0))
        pltpu.sync_copy(ubuf, u_hbm)


def kernel(positions, box_vectors, lig_indices, ref_com, k):
    n = positions.shape[0]
    nlig = lig_indices.shape[0]
    flt = 3 * n
    params = jnp.concatenate([
        box_vectors.astype(jnp.float32).reshape(9),
        ref_com.astype(jnp.float32).reshape(3),
        k.astype(jnp.float32).reshape(1),
        jnp.zeros((3,), jnp.float32),
    ])
    posf = positions.reshape(flt)
    n96 = 3 * nlig

    mesh = plsc.VectorSubcoreMesh(core_axis_name="c", subcore_axis_name="s",
                                  num_cores=NC, num_subcores=NS)
    f = pl.kernel(
        functools.partial(_sc_body, nlig, flt),
        out_type=[
            jax.ShapeDtypeStruct((L,), jnp.float32),
            jax.ShapeDtypeStruct((flt,), jnp.float32),
        ],
        mesh=mesh,
        compiler_params=pltpu.CompilerParams(needs_layout_passes=False),
        scratch_types=[
            pltpu.VMEM((nlig,), jnp.int32),
            pltpu.VMEM((n96,), jnp.int32),
            pltpu.VMEM((n96,), jnp.float32),
            pltpu.VMEM((n96,), jnp.int32),
            pltpu.VMEM((n96,), jnp.float32),
            pltpu.VMEM((L,), jnp.float32),
            pltpu.VMEM((W,), jnp.float32),
            pltpu.SemaphoreType.DMA,
            pltpu.SemaphoreType.DMA,
            pltpu.SemaphoreType.DMA,
        ],
    )
    u16, outf = f(posf, lig_indices, params)
    return (u16[0], outf.reshape(n, 3))
